# speedup vs baseline: 1.5359x; 1.5359x over previous
import jax
import jax.numpy as jnp
from jax import lax
from jax.experimental import pallas as pl
from jax.experimental.pallas import tpu as pltpu
from jax.experimental.pallas import tpu_sc as plsc

_B, _S, _D = 4, 8192, 1024
_M = 2048
_NC, _NS = 2, 16
_NW = _NC * _NS
_RPW = (_B * _M) // _NW
_K = 32
_NCHUNK = _RPW // _K
_WPB = _M // _RPW


def _body(x_hbm, idx_hbm, out_hbm, idx_v, buf0, buf1, sem0, sem1):
    wid = lax.axis_index("s") * _NC + lax.axis_index("c")
    base = wid * _RPW
    off = (wid // _WPB) * _S
    pltpu.sync_copy(idx_hbm.at[pl.ds(base, _RPW)], idx_v)
    for i in range(_RPW // 16):
        sl = pl.ds(i * 16, 16)
        idx_v[sl] = idx_v[sl] + off
    bufs = (buf0, buf1)
    sems = (sem0, sem1)
    cps = [None, None]
    cps[0] = pltpu.async_copy(x_hbm.at[idx_v.at[pl.ds(0, _K)]], bufs[0], sems[0])
    for c in range(_NCHUNK):
        nxt = c + 1
        if nxt < _NCHUNK:
            cps[nxt % 2] = pltpu.async_copy(
                x_hbm.at[idx_v.at[pl.ds(nxt * _K, _K)]], bufs[nxt % 2], sems[nxt % 2])
        cps[c % 2].wait()
        pltpu.sync_copy(bufs[c % 2], out_hbm.at[pl.ds(base + c * _K, _K)])


@jax.jit
def _gather_flat(xf, idxf):
    mesh = plsc.VectorSubcoreMesh(core_axis_name="c", subcore_axis_name="s")
    f = pl.kernel(
        _body,
        mesh=mesh,
        out_type=jax.ShapeDtypeStruct((_B * _M, _D), jnp.float32),
        scratch_types=[
            pltpu.VMEM((_RPW,), jnp.int32),
            pltpu.VMEM((_K, _D), jnp.float32),
            pltpu.VMEM((_K, _D), jnp.float32),
            pltpu.SemaphoreType.DMA,
            pltpu.SemaphoreType.DMA,
        ],
    )
    return f(xf, idxf)


def kernel(x, indices_to_keep):
    idxf = indices_to_keep.astype(jnp.int32).reshape(_B * _M)
    xf = x.reshape(_B * _S, _D)
    out = _gather_flat(xf, idxf)
    return out.reshape(_B, _M, _D)


# speedup vs baseline: 1.5551x; 1.0125x over previous
import jax
import jax.numpy as jnp
from jax import lax
from jax.experimental import pallas as pl
from jax.experimental.pallas import tpu as pltpu
from jax.experimental.pallas import tpu_sc as plsc

_B, _S, _D = 4, 8192, 1024
_M = 2048
_NC, _NS = 2, 16
_NW = _NC * _NS
_RPW = (_B * _M) // _NW
_K = 32
_NCHUNK = _RPW // _K
_WPB = _M // _RPW


_NBUF = 3


def _body(x_hbm, idx_hbm, out_hbm, idx_v,
          buf0, buf1, buf2, gsem0, gsem1, gsem2, ssem0, ssem1, ssem2):
    wid = lax.axis_index("s") * _NC + lax.axis_index("c")
    base = wid * _RPW
    off = (wid // _WPB) * _S
    pltpu.sync_copy(idx_hbm.at[pl.ds(base, _RPW)], idx_v)
    for i in range(_RPW // 16):
        sl = pl.ds(i * 16, 16)
        idx_v[sl] = idx_v[sl] + off
    bufs = (buf0, buf1, buf2)
    gsems = (gsem0, gsem1, gsem2)
    ssems = (ssem0, ssem1, ssem2)
    gcps = [None] * _NBUF
    scps = [None] * _NBUF

    def gather(c):
        s = c % _NBUF
        return pltpu.async_copy(
            x_hbm.at[idx_v.at[pl.ds(c * _K, _K)]], bufs[s], gsems[s])

    for c in range(min(_NBUF, _NCHUNK)):
        gcps[c % _NBUF] = gather(c)
    for c in range(_NCHUNK):
        s = c % _NBUF
        gcps[s].wait()
        scps[s] = pltpu.async_copy(bufs[s], out_hbm.at[pl.ds(base + c * _K, _K)],
                                   ssems[s])
        nxt = c + _NBUF
        if nxt < _NCHUNK:
            scps[s].wait()
            gcps[s] = gather(nxt)
    for c in range(max(0, _NCHUNK - _NBUF), _NCHUNK):
        scps[c % _NBUF].wait()


@jax.jit
def _gather_flat(xf, idxf):
    mesh = plsc.VectorSubcoreMesh(core_axis_name="c", subcore_axis_name="s")
    f = pl.kernel(
        _body,
        mesh=mesh,
        out_type=jax.ShapeDtypeStruct((_B * _M, _D), jnp.float32),
        scratch_types=[
            pltpu.VMEM((_RPW,), jnp.int32),
            pltpu.VMEM((_K, _D), jnp.float32),
            pltpu.VMEM((_K, _D), jnp.float32),
            pltpu.VMEM((_K, _D), jnp.float32),
            pltpu.SemaphoreType.DMA,
            pltpu.SemaphoreType.DMA,
            pltpu.SemaphoreType.DMA,
            pltpu.SemaphoreType.DMA,
            pltpu.SemaphoreType.DMA,
            pltpu.SemaphoreType.DMA,
        ],
    )
    return f(xf, idxf)


def kernel(x, indices_to_keep):
    idxf = indices_to_keep.astype(jnp.int32).reshape(_B * _M)
    xf = x.reshape(_B * _S, _D)
    out = _gather_flat(xf, idxf)
    return out.reshape(_B, _M, _D)


# speedup vs baseline: 1.5706x; 1.0100x over previous
import jax
import jax.numpy as jnp
from jax import lax
from jax.experimental import pallas as pl
from jax.experimental.pallas import tpu as pltpu
from jax.experimental.pallas import tpu_sc as plsc

_B, _S, _D = 4, 8192, 1024
_M = 2048
_NC, _NS = 2, 16
_NW = _NC * _NS
_RPW = (_B * _M) // _NW
_K = 16
_NCHUNK = _RPW // _K
_NBUF = 4
_NGROUP = _NCHUNK // _NBUF
_WPB = _M // _RPW


def _body(x_hbm, idx_hbm, out_hbm, idx_v,
          buf0, buf1, buf2, buf3,
          gsem0, gsem1, gsem2, gsem3, ssem0, ssem1, ssem2, ssem3):
    wid = lax.axis_index("s") * _NC + lax.axis_index("c")
    base = wid * _RPW
    off = (wid // _WPB) * _S
    pltpu.sync_copy(idx_hbm.at[pl.ds(base, _RPW)], idx_v)
    bufs = (buf0, buf1, buf2, buf3)
    gsems = (gsem0, gsem1, gsem2, gsem3)
    ssems = (ssem0, ssem1, ssem2, ssem3)

    def start_gather(c, s):
        iv = idx_v[pl.ds(c * _K, _K)] + off
        pltpu.async_copy(x_hbm.at[iv], bufs[s], gsems[s])

    def wait_gather(s):
        pltpu.make_async_copy(x_hbm.at[pl.ds(0, _K)], bufs[s], gsems[s]).wait()

    def start_store(c, s):
        pltpu.async_copy(bufs[s], out_hbm.at[pl.ds(base + c * _K, _K)], ssems[s])

    def wait_store(c, s):
        pltpu.make_async_copy(bufs[s], out_hbm.at[pl.ds(base + c * _K, _K)],
                              ssems[s]).wait()

    for s in range(_NBUF):
        start_gather(s, s)

    def group(g, carry):
        for s in range(_NBUF):
            c = g * _NBUF + s
            wait_gather(s)
            start_store(c, s)
            wait_store(c, s)
            nxt = c + _NBUF
            @pl.when(nxt < _NCHUNK)
            def _():
                start_gather(nxt, s)
        return carry

    lax.fori_loop(0, _NGROUP, group, 0)


@jax.jit
def _gather_flat(xf, idxf):
    mesh = plsc.VectorSubcoreMesh(core_axis_name="c", subcore_axis_name="s")
    f = pl.kernel(
        _body,
        mesh=mesh,
        out_type=jax.ShapeDtypeStruct((_B * _M, _D), jnp.float32),
        scratch_types=[
            pltpu.VMEM((_RPW,), jnp.int32),
            pltpu.VMEM((_K, _D), jnp.float32),
            pltpu.VMEM((_K, _D), jnp.float32),
            pltpu.VMEM((_K, _D), jnp.float32),
            pltpu.VMEM((_K, _D), jnp.float32),
            pltpu.SemaphoreType.DMA,
            pltpu.SemaphoreType.DMA,
            pltpu.SemaphoreType.DMA,
            pltpu.SemaphoreType.DMA,
            pltpu.SemaphoreType.DMA,
            pltpu.SemaphoreType.DMA,
            pltpu.SemaphoreType.DMA,
            pltpu.SemaphoreType.DMA,
        ],
    )
    return f(xf, idxf)


def kernel(x, indices_to_keep):
    idxf = indices_to_keep.astype(jnp.int32).reshape(_B * _M)
    xf = x.reshape(_B * _S, _D)
    out = _gather_flat(xf, idxf)
    return out.reshape(_B, _M, _D)
